# baseline (speedup 1.0000x reference)
import functools

import jax
import jax.numpy as jnp
from jax import lax
from jax.experimental import pallas as pl
from jax.experimental.pallas import tpu as pltpu
from jax.experimental.pallas import tpu_sc as plsc

N = 10000
E = 320000
D = 128
H = 256
DE = 16

NC = 2
NS = 16
B = 80
B1 = 40
B3 = 80
IC = 10
IC3 = 5
OC = 25
OC1 = 50
NH = N // 2
NPH = 5120
JUNK = NPH - 1
RPTH = NPH // NS
NP = 10240
RPT = NP // NS
BN = 1000
BE = 8000

_f32 = jnp.float32


@functools.partial(
    pl.kernel,
    out_type=(jax.ShapeDtypeStruct((NC, NPH, 128), _f32),
              jax.ShapeDtypeStruct((NC, NPH, 128), _f32)),
    mesh=plsc.VectorSubcoreMesh(core_axis_name="c", subcore_axis_name="s"),
    scratch_types=[
        pltpu.VMEM_SHARED((NPH, 128), _f32),
        pltpu.VMEM((IC, B1), jnp.int32),
        pltpu.VMEM((IC, B1), jnp.int32),
        pltpu.VMEM((B1, 128), _f32),
        pltpu.VMEM((B1, 128), _f32),
        pltpu.SemaphoreType.DMA,
        pltpu.SemaphoreType.DMA,
    ],
)
def _sc_layer1(x, srcT, dstC, z128, ones128, cnt_hbm, agg_hbm,
               acc_sh, idx_s, idx_d, rows0, rows1, sem0, sem1):
    c = lax.axis_index("c")
    s = lax.axis_index("s")
    r0 = s * RPTH
    pltpu.sync_copy(z128.at[pl.ds(r0, RPTH), :], acc_sh.at[pl.ds(r0, RPTH), :])
    pltpu.sync_copy(ones128, rows0)
    plsc.subcore_barrier()

    def outer_cnt(o, carry):
        pltpu.sync_copy(dstC.at[c, s, o], idx_d)
        cps = [pltpu.async_copy(rows0, acc_sh.at[idx_d.at[k]], sem0 if
                                k % 2 == 0 else sem1, add=True)
               for k in range(IC)]
        for cp in cps:
            cp.wait()
        return carry

    lax.fori_loop(0, OC1, outer_cnt, 0)
    plsc.subcore_barrier()
    pltpu.sync_copy(acc_sh.at[pl.ds(r0, RPTH), :],
                    cnt_hbm.at[c, pl.ds(r0, RPTH), :])
    pltpu.sync_copy(z128.at[pl.ds(r0, RPTH), :], acc_sh.at[pl.ds(r0, RPTH), :])
    plsc.subcore_barrier()

    def outer_agg(o, carry):
        pltpu.sync_copy(srcT.at[s, o], idx_s)
        pltpu.sync_copy(dstC.at[c, s, o], idx_d)
        bufs = (rows0, rows1)
        sems = (sem0, sem1)
        g = pltpu.async_copy(x.at[idx_s.at[0]], bufs[0], sems[0])
        for k in range(IC):
            nxt = None
            if k + 1 < IC:
                nxt = pltpu.async_copy(x.at[idx_s.at[k + 1]],
                                       bufs[(k + 1) % 2], sems[(k + 1) % 2])
            g.wait()
            pltpu.sync_copy(bufs[k % 2], acc_sh.at[idx_d.at[k]], add=True)
            g = nxt
        return carry

    lax.fori_loop(0, OC1, outer_agg, 0)
    plsc.subcore_barrier()
    pltpu.sync_copy(acc_sh.at[pl.ds(r0, RPTH), :],
                    agg_hbm.at[c, pl.ds(r0, RPTH), :])


@functools.partial(
    pl.kernel,
    out_type=jax.ShapeDtypeStruct((NC, NP, 128), _f32),
    mesh=plsc.VectorSubcoreMesh(core_axis_name="c", subcore_axis_name="s"),
    scratch_types=[
        pltpu.VMEM_SHARED((NP, 128), _f32),
        pltpu.VMEM((IC, B), jnp.int32),
        pltpu.VMEM((IC, B), jnp.int32),
        pltpu.VMEM((B, 128), _f32),
        pltpu.VMEM((B, 128), _f32),
        pltpu.SemaphoreType.DMA,
        pltpu.SemaphoreType.DMA,
    ],
)
def _sc_layer2(h1f, srcMc, dstT, z128, agg_hbm,
               agg_sh, idx_s, idx_d, rows0, rows1, sem0, sem1):
    c = lax.axis_index("c")
    s = lax.axis_index("s")
    r0 = s * RPT
    pltpu.sync_copy(z128.at[pl.ds(r0, RPT), :], agg_sh.at[pl.ds(r0, RPT), :])
    plsc.subcore_barrier()

    def outer(o, carry):
        pltpu.sync_copy(srcMc.at[c, s, o], idx_s)
        pltpu.sync_copy(dstT.at[s, o], idx_d)
        bufs = (rows0, rows1)
        sems = (sem0, sem1)
        g = pltpu.async_copy(h1f.at[idx_s.at[0]], bufs[0], sems[0])
        for k in range(IC):
            nxt = None
            if k + 1 < IC:
                nxt = pltpu.async_copy(h1f.at[idx_s.at[k + 1]],
                                       bufs[(k + 1) % 2], sems[(k + 1) % 2])
            g.wait()
            pltpu.sync_copy(bufs[k % 2], agg_sh.at[idx_d.at[k]], add=True)
            g = nxt
        return carry

    lax.fori_loop(0, OC, outer, 0)
    plsc.subcore_barrier()
    pltpu.sync_copy(agg_sh.at[pl.ds(r0, RPT), :],
                    agg_hbm.at[c, pl.ds(r0, RPT), :])


@functools.partial(
    pl.kernel,
    out_type=jax.ShapeDtypeStruct((E, 2 * H + DE), _f32),
    mesh=plsc.VectorSubcoreMesh(core_axis_name="c", subcore_axis_name="s"),
    scratch_types=[
        pltpu.VMEM((IC3, B3), jnp.int32),
        pltpu.VMEM((IC3, B3), jnp.int32),
        pltpu.VMEM((B3, H), _f32),
        pltpu.VMEM((B3, H), _f32),
        pltpu.VMEM((B3, H), _f32),
        pltpu.VMEM((B3, H), _f32),
        pltpu.SemaphoreType.DMA,
        pltpu.SemaphoreType.DMA,
        pltpu.SemaphoreType.DMA,
        pltpu.SemaphoreType.DMA,
    ],
)
def _sc_out(g, srcW, dstW, ea, out_hbm,
            idx_s, idx_d, bs0, bs1, bd0, bd1, ss0, ss1, sd0, sd1):
    c = lax.axis_index("c")
    s = lax.axis_index("s")
    w = s * NC + c

    def outer(o, carry):
        pltpu.sync_copy(srcW.at[w, o], idx_s)
        pltpu.sync_copy(dstW.at[w, o], idx_d)
        bS = (bs0, bs1)
        bD = (bd0, bd1)
        sS = (ss0, ss1)
        sD = (sd0, sd1)
        gs = pltpu.async_copy(g.at[idx_s.at[0]], bS[0], sS[0])
        gd = pltpu.async_copy(g.at[idx_d.at[0]], bD[0], sD[0])
        for k in range(IC3):
            e0 = (w * OC * IC3 + o * IC3 + k) * B3
            ns = nd = None
            if k + 1 < IC3:
                p = (k + 1) % 2
                ns = pltpu.async_copy(g.at[idx_s.at[k + 1]], bS[p], sS[p])
                nd = pltpu.async_copy(g.at[idx_d.at[k + 1]], bD[p], sD[p])
            pltpu.sync_copy(ea.at[pl.ds(e0, B3), :],
                            out_hbm.at[pl.ds(e0, B3), pl.ds(2 * H, DE)])
            gs.wait()
            gd.wait()
            pltpu.sync_copy(bS[k % 2], out_hbm.at[pl.ds(e0, B3), pl.ds(0, H)])
            pltpu.sync_copy(bD[k % 2], out_hbm.at[pl.ds(e0, B3), pl.ds(H, H)])
            gs, gd = ns, nd
        return carry

    lax.fori_loop(0, OC, outer, 0)


def _agg_map(i):
    return (i // 5, i % 5, 0)


def _tc1_body(agg_ref, cnt_ref, x_ref, w1l_ref, b1_ref, w1r_ref, o_ref):
    inv = 1.0 / jnp.maximum(cnt_ref[0][:, 0:1], 1.0)
    h = (jnp.dot(agg_ref[0] * inv, w1l_ref[...], preferred_element_type=_f32)
         + jnp.dot(x_ref[...], w1r_ref[...], preferred_element_type=_f32)
         + b1_ref[...])
    h = jnp.maximum(h, 0.0)
    o_ref[0] = h[:, 0:128]
    o_ref[1] = h[:, 128:256]


def _tc1(agg1, cnt, x, w1l, b1, w1r):
    return pl.pallas_call(
        _tc1_body,
        grid=(N // BN,),
        in_specs=[
            pl.BlockSpec((1, BN, 128), _agg_map),
            pl.BlockSpec((1, BN, 128), _agg_map),
            pl.BlockSpec((BN, D), lambda i: (i, 0)),
            pl.BlockSpec((D, H), lambda i: (0, 0)),
            pl.BlockSpec((1, H), lambda i: (0, 0)),
            pl.BlockSpec((D, H), lambda i: (0, 0)),
        ],
        out_specs=pl.BlockSpec((NC, BN, 128), lambda i: (0, i, 0)),
        out_shape=jax.ShapeDtypeStruct((NC, N, 128), _f32),
    )(agg1, cnt, x, w1l, b1, w1r)


def _tc2_body(agg2_ref, cnt_ref, h1_ref, w2l_ref, b2_ref, w2r_ref, o_ref):
    inv = 1.0 / jnp.maximum(cnt_ref[0][:, 0:1], 1.0)
    t = (jnp.dot(agg2_ref[0] * inv, w2l_ref[0:128, :],
                 preferred_element_type=_f32)
         + jnp.dot(agg2_ref[1] * inv, w2l_ref[128:256, :],
                   preferred_element_type=_f32)
         + jnp.dot(h1_ref[0], w2r_ref[0:128, :], preferred_element_type=_f32)
         + jnp.dot(h1_ref[1], w2r_ref[128:256, :],
                   preferred_element_type=_f32)
         + b2_ref[...])
    o_ref[...] = 1.0 / (1.0 + jnp.exp(-t))


def _tc2(agg2, cnt, h1s, w2l, b2, w2r):
    return pl.pallas_call(
        _tc2_body,
        grid=(N // BN,),
        in_specs=[
            pl.BlockSpec((NC, BN, 128), lambda i: (0, i, 0)),
            pl.BlockSpec((1, BN, 128), _agg_map),
            pl.BlockSpec((NC, BN, 128), lambda i: (0, i, 0)),
            pl.BlockSpec((H, H), lambda i: (0, 0)),
            pl.BlockSpec((1, H), lambda i: (0, 0)),
            pl.BlockSpec((H, H), lambda i: (0, 0)),
        ],
        out_specs=pl.BlockSpec((BN, H), lambda i: (i, 0)),
        out_shape=jax.ShapeDtypeStruct((N, H), _f32),
    )(agg2, cnt, h1s, w2l, b2, w2r)


def _tc3_body(ea_ref, o_ref):
    o_ref[...] = 1.0 / (1.0 + jnp.exp(-ea_ref[...]))


def _tc3(edge_attr):
    return pl.pallas_call(
        _tc3_body,
        grid=(E // BE,),
        in_specs=[pl.BlockSpec((BE, DE), lambda i: (i, 0))],
        out_specs=pl.BlockSpec((BE, DE), lambda i: (i, 0)),
        out_shape=jax.ShapeDtypeStruct((E, DE), _f32),
    )(edge_attr)


def kernel(x, edge_index, edge_attr, W1l, b1, W1r, W2l, b2, W2r):
    src = edge_index[0]
    dst = edge_index[1]
    srcT = src.reshape(NS, OC1, IC, B1)
    srcT2 = src.reshape(NS, OC, IC, B)
    srcMc = jnp.stack([srcT2, srcT2 + N])
    dstT = dst.reshape(NS, OC, IC, B)
    dstC = jnp.stack([jnp.where(dst < NH, dst, JUNK),
                      jnp.where(dst >= NH, dst - NH, JUNK)])
    dstC = dstC.reshape(NC, NS, OC1, IC, B1)
    srcW = src.reshape(NC * NS, OC, IC3, B3)
    dstW = dst.reshape(NC * NS, OC, IC3, B3)
    z128 = jnp.zeros((NP, 128), _f32)
    ones128 = jnp.ones((B1, 128), _f32)

    cnt, agg1 = _sc_layer1(x, srcT, dstC, z128, ones128)
    h1s = _tc1(agg1, cnt, x, W1l, b1.reshape(1, H), W1r)
    agg2 = _sc_layer2(h1s.reshape(2 * N, 128), srcMc, dstT, z128)
    g = _tc2(agg2, cnt, h1s, W2l, b2.reshape(1, H), W2r)
    ea = _tc3(edge_attr)
    out = _sc_out(g, srcW, dstW, ea)
    return out


# speedup vs baseline: 2.9338x; 2.9338x over previous
import functools

import jax
import jax.numpy as jnp
from jax import lax
from jax.experimental import pallas as pl
from jax.experimental.pallas import tpu as pltpu
from jax.experimental.pallas import tpu_sc as plsc

N = 10000
E = 320000
D = 128
H = 256
DE = 16

NC = 2
NS = 16
B = 80
B3 = 40
IC = 10
OC = 25
NH = N // 2
NPH = 5120
JUNK = NPH - 1
RPTH = NPH // NS
NP = 10240
RPT = NP // NS
BN = 1000
BE = 8000

_f32 = jnp.float32


@functools.partial(
    pl.kernel,
    out_type=(jax.ShapeDtypeStruct((NC, NPH, 128), _f32),
              jax.ShapeDtypeStruct((NC, NPH, 128), _f32)),
    mesh=plsc.VectorSubcoreMesh(core_axis_name="c", subcore_axis_name="s"),
    scratch_types=[
        pltpu.VMEM_SHARED((NPH, 128), _f32),
        pltpu.VMEM((IC, B), jnp.int32),
        pltpu.VMEM((IC, B), jnp.int32),
        pltpu.VMEM((B, 128), _f32),
        pltpu.VMEM((B, 128), _f32),
        pltpu.SemaphoreType.DMA,
        pltpu.SemaphoreType.DMA,
    ],
)
def _sc_layer1(x, srcT, dstC, z128, ones128, cnt_hbm, agg_hbm,
               acc_sh, idx_s, idx_d, rows0, rows1, sem0, sem1):
    c = lax.axis_index("c")
    s = lax.axis_index("s")
    r0 = s * RPTH
    pltpu.sync_copy(z128.at[pl.ds(r0, RPTH), :], acc_sh.at[pl.ds(r0, RPTH), :])
    pltpu.sync_copy(ones128, rows0)
    plsc.subcore_barrier()

    def outer_cnt(o, carry):
        pltpu.sync_copy(dstC.at[c, s, o], idx_d)
        cps = [pltpu.async_copy(rows0, acc_sh.at[idx_d.at[k]], sem0 if
                                k % 2 == 0 else sem1, add=True)
               for k in range(IC)]
        for cp in cps:
            cp.wait()
        return carry

    lax.fori_loop(0, OC, outer_cnt, 0)
    plsc.subcore_barrier()
    pltpu.sync_copy(acc_sh.at[pl.ds(r0, RPTH), :],
                    cnt_hbm.at[c, pl.ds(r0, RPTH), :])
    pltpu.sync_copy(z128.at[pl.ds(r0, RPTH), :], acc_sh.at[pl.ds(r0, RPTH), :])
    plsc.subcore_barrier()

    def outer_agg(o, carry):
        pltpu.sync_copy(srcT.at[s, o], idx_s)
        pltpu.sync_copy(dstC.at[c, s, o], idx_d)
        bufs = (rows0, rows1)
        sems = (sem0, sem1)
        g = pltpu.async_copy(x.at[idx_s.at[0]], bufs[0], sems[0])
        for k in range(IC):
            nxt = None
            if k + 1 < IC:
                nxt = pltpu.async_copy(x.at[idx_s.at[k + 1]],
                                       bufs[(k + 1) % 2], sems[(k + 1) % 2])
            g.wait()
            pltpu.sync_copy(bufs[k % 2], acc_sh.at[idx_d.at[k]], add=True)
            g = nxt
        return carry

    lax.fori_loop(0, OC, outer_agg, 0)
    plsc.subcore_barrier()
    pltpu.sync_copy(acc_sh.at[pl.ds(r0, RPTH), :],
                    agg_hbm.at[c, pl.ds(r0, RPTH), :])


@functools.partial(
    pl.kernel,
    out_type=jax.ShapeDtypeStruct((NC, NP, 128), _f32),
    mesh=plsc.VectorSubcoreMesh(core_axis_name="c", subcore_axis_name="s"),
    scratch_types=[
        pltpu.VMEM_SHARED((NP, 128), _f32),
        pltpu.VMEM((IC, B), jnp.int32),
        pltpu.VMEM((IC, B), jnp.int32),
        pltpu.VMEM((B, 128), _f32),
        pltpu.VMEM((B, 128), _f32),
        pltpu.SemaphoreType.DMA,
        pltpu.SemaphoreType.DMA,
    ],
)
def _sc_layer2(h1f, srcMc, dstT, z128, agg_hbm,
               agg_sh, idx_s, idx_d, rows0, rows1, sem0, sem1):
    c = lax.axis_index("c")
    s = lax.axis_index("s")
    r0 = s * RPT
    pltpu.sync_copy(z128.at[pl.ds(r0, RPT), :], agg_sh.at[pl.ds(r0, RPT), :])
    plsc.subcore_barrier()

    def outer(o, carry):
        pltpu.sync_copy(srcMc.at[c, s, o], idx_s)
        pltpu.sync_copy(dstT.at[s, o], idx_d)
        bufs = (rows0, rows1)
        sems = (sem0, sem1)
        g = pltpu.async_copy(h1f.at[idx_s.at[0]], bufs[0], sems[0])
        for k in range(IC):
            nxt = None
            if k + 1 < IC:
                nxt = pltpu.async_copy(h1f.at[idx_s.at[k + 1]],
                                       bufs[(k + 1) % 2], sems[(k + 1) % 2])
            g.wait()
            pltpu.sync_copy(bufs[k % 2], agg_sh.at[idx_d.at[k]], add=True)
            g = nxt
        return carry

    lax.fori_loop(0, OC, outer, 0)
    plsc.subcore_barrier()
    pltpu.sync_copy(agg_sh.at[pl.ds(r0, RPT), :],
                    agg_hbm.at[c, pl.ds(r0, RPT), :])


@functools.partial(
    pl.kernel,
    out_type=jax.ShapeDtypeStruct((E, 2 * H + DE), _f32),
    mesh=plsc.VectorSubcoreMesh(core_axis_name="c", subcore_axis_name="s"),
    scratch_types=[
        pltpu.VMEM((IC, B3), jnp.int32),
        pltpu.VMEM((IC, B3), jnp.int32),
        pltpu.VMEM((B3, H), _f32),
        pltpu.VMEM((B3, H), _f32),
        pltpu.VMEM((B3, H), _f32),
        pltpu.VMEM((B3, H), _f32),
        pltpu.VMEM((B3, DE), _f32),
        pltpu.VMEM((B3, DE), _f32),
        pltpu.SemaphoreType.DMA,
        pltpu.SemaphoreType.DMA,
        pltpu.SemaphoreType.DMA,
        pltpu.SemaphoreType.DMA,
        pltpu.SemaphoreType.DMA,
        pltpu.SemaphoreType.DMA,
        pltpu.SemaphoreType.DMA,
        pltpu.SemaphoreType.DMA,
        pltpu.SemaphoreType.DMA,
        pltpu.SemaphoreType.DMA,
    ],
)
def _sc_out(g, srcW, dstW, ea, out_hbm,
            idx_s, idx_d, bs0, bs1, bd0, bd1, be0, be1,
            ss0, ss1, sd0, sd1, ts0, ts1, td0, td1, te0, te1):
    c = lax.axis_index("c")
    s = lax.axis_index("s")
    w = s * NC + c

    def outer(o, carry):
        pltpu.sync_copy(srcW.at[w, o], idx_s)
        pltpu.sync_copy(dstW.at[w, o], idx_d)
        bS = (bs0, bs1)
        bD = (bd0, bd1)
        bE = (be0, be1)
        sS = (ss0, ss1)
        sD = (sd0, sd1)
        tS = (ts0, ts1)
        tD = (td0, td1)
        tE = (te0, te1)
        gs = pltpu.async_copy(g.at[idx_s.at[0]], bS[0], sS[0])
        gd = pltpu.async_copy(g.at[idx_d.at[0]], bD[0], sD[0])
        stores = [None, None]
        for k in range(IC):
            p = k % 2
            e0 = (w * OC * IC + o * IC + k) * B3
            ns = nd = None
            if k + 1 < IC:
                q = (k + 1) % 2
                if stores[q] is not None:
                    for d in stores[q]:
                        d.wait()
                    stores[q] = None
                ns = pltpu.async_copy(g.at[idx_s.at[k + 1]], bS[q], sS[q])
                nd = pltpu.async_copy(g.at[idx_d.at[k + 1]], bD[q], sD[q])
            pltpu.sync_copy(ea.at[pl.ds(e0, B3), :], bE[p])
            gs.wait()
            gd.wait()
            stores[p] = (
                pltpu.async_copy(bS[p],
                                 out_hbm.at[pl.ds(e0, B3), pl.ds(0, H)],
                                 tS[p]),
                pltpu.async_copy(bD[p],
                                 out_hbm.at[pl.ds(e0, B3), pl.ds(H, H)],
                                 tD[p]),
                pltpu.async_copy(bE[p],
                                 out_hbm.at[pl.ds(e0, B3), pl.ds(2 * H, DE)],
                                 tE[p]),
            )
            gs, gd = ns, nd
        for par in (0, 1):
            if stores[par] is not None:
                for d in stores[par]:
                    d.wait()
        return carry

    lax.fori_loop(0, OC, outer, 0)


def _agg_map(i):
    return (i // 5, i % 5, 0)


def _tc1_body(agg_ref, cnt_ref, x_ref, w1l_ref, b1_ref, w1r_ref, o_ref):
    inv = 1.0 / jnp.maximum(cnt_ref[0][:, 0:1], 1.0)
    h = (jnp.dot(agg_ref[0] * inv, w1l_ref[...], preferred_element_type=_f32)
         + jnp.dot(x_ref[...], w1r_ref[...], preferred_element_type=_f32)
         + b1_ref[...])
    h = jnp.maximum(h, 0.0)
    o_ref[0] = h[:, 0:128]
    o_ref[1] = h[:, 128:256]


def _tc1(agg1, cnt, x, w1l, b1, w1r):
    return pl.pallas_call(
        _tc1_body,
        grid=(N // BN,),
        in_specs=[
            pl.BlockSpec((1, BN, 128), _agg_map),
            pl.BlockSpec((1, BN, 128), _agg_map),
            pl.BlockSpec((BN, D), lambda i: (i, 0)),
            pl.BlockSpec((D, H), lambda i: (0, 0)),
            pl.BlockSpec((1, H), lambda i: (0, 0)),
            pl.BlockSpec((D, H), lambda i: (0, 0)),
        ],
        out_specs=pl.BlockSpec((NC, BN, 128), lambda i: (0, i, 0)),
        out_shape=jax.ShapeDtypeStruct((NC, N, 128), _f32),
    )(agg1, cnt, x, w1l, b1, w1r)


def _tc2_body(agg2_ref, cnt_ref, h1_ref, w2l_ref, b2_ref, w2r_ref, o_ref):
    inv = 1.0 / jnp.maximum(cnt_ref[0][:, 0:1], 1.0)
    t = (jnp.dot(agg2_ref[0] * inv, w2l_ref[0:128, :],
                 preferred_element_type=_f32)
         + jnp.dot(agg2_ref[1] * inv, w2l_ref[128:256, :],
                   preferred_element_type=_f32)
         + jnp.dot(h1_ref[0], w2r_ref[0:128, :], preferred_element_type=_f32)
         + jnp.dot(h1_ref[1], w2r_ref[128:256, :],
                   preferred_element_type=_f32)
         + b2_ref[...])
    o_ref[...] = 1.0 / (1.0 + jnp.exp(-t))


def _tc2(agg2, cnt, h1s, w2l, b2, w2r):
    return pl.pallas_call(
        _tc2_body,
        grid=(N // BN,),
        in_specs=[
            pl.BlockSpec((NC, BN, 128), lambda i: (0, i, 0)),
            pl.BlockSpec((1, BN, 128), _agg_map),
            pl.BlockSpec((NC, BN, 128), lambda i: (0, i, 0)),
            pl.BlockSpec((H, H), lambda i: (0, 0)),
            pl.BlockSpec((1, H), lambda i: (0, 0)),
            pl.BlockSpec((H, H), lambda i: (0, 0)),
        ],
        out_specs=pl.BlockSpec((BN, H), lambda i: (i, 0)),
        out_shape=jax.ShapeDtypeStruct((N, H), _f32),
    )(agg2, cnt, h1s, w2l, b2, w2r)


def _tc3_body(ea_ref, o_ref):
    o_ref[...] = 1.0 / (1.0 + jnp.exp(-ea_ref[...]))


def _tc3(edge_attr):
    return pl.pallas_call(
        _tc3_body,
        grid=(E // BE,),
        in_specs=[pl.BlockSpec((BE, DE), lambda i: (i, 0))],
        out_specs=pl.BlockSpec((BE, DE), lambda i: (i, 0)),
        out_shape=jax.ShapeDtypeStruct((E, DE), _f32),
    )(edge_attr)


def kernel(x, edge_index, edge_attr, W1l, b1, W1r, W2l, b2, W2r):
    src = edge_index[0]
    dst = edge_index[1]
    srcT = src.reshape(NS, OC, IC, B)
    srcMc = jnp.stack([srcT, srcT + N])
    dstT = dst.reshape(NS, OC, IC, B)
    dstC = jnp.stack([jnp.where(dst < NH, dst, JUNK),
                      jnp.where(dst >= NH, dst - NH, JUNK)])
    dstC = dstC.reshape(NC, NS, OC, IC, B)
    srcW = src.reshape(NC * NS, OC, IC, B3)
    dstW = dst.reshape(NC * NS, OC, IC, B3)
    z128 = jnp.zeros((NP, 128), _f32)
    ones128 = jnp.ones((B, 128), _f32)

    cnt, agg1 = _sc_layer1(x, srcT, dstC, z128, ones128)
    h1s = _tc1(agg1, cnt, x, W1l, b1.reshape(1, H), W1r)
    agg2 = _sc_layer2(h1s.reshape(2 * N, 128), srcMc, dstT, z128)
    g = _tc2(agg2, cnt, h1s, W2l, b2.reshape(1, H), W2r)
    ea = _tc3(edge_attr)
    out = _sc_out(g, srcW, dstW, ea)
    return out


# speedup vs baseline: 3.0039x; 1.0239x over previous
import functools

import jax
import jax.numpy as jnp
from jax import lax
from jax.experimental import pallas as pl
from jax.experimental.pallas import tpu as pltpu
from jax.experimental.pallas import tpu_sc as plsc

N = 10000
E = 320000
D = 128
H = 256
DE = 16

NC = 2
NS = 16
B = 80
B3 = 80
IC = 10
IC3 = 5
OC = 25
NH = N // 2
NPH = 5120
JUNK = NPH - 1
RPTH = NPH // NS
NP = 10240
RPT = NP // NS
BN = 1000
BE = 8000

_f32 = jnp.float32


@functools.partial(
    pl.kernel,
    out_type=(jax.ShapeDtypeStruct((NC, NPH, 128), _f32),
              jax.ShapeDtypeStruct((NC, NPH, 128), _f32)),
    mesh=plsc.VectorSubcoreMesh(core_axis_name="c", subcore_axis_name="s"),
    scratch_types=[
        pltpu.VMEM_SHARED((NPH, 128), _f32),
        pltpu.VMEM((IC, B), jnp.int32),
        pltpu.VMEM((IC, B), jnp.int32),
        pltpu.VMEM((B, 128), _f32),
        pltpu.VMEM((B, 128), _f32),
        pltpu.SemaphoreType.DMA,
        pltpu.SemaphoreType.DMA,
    ],
)
def _sc_layer1(x, srcT, dstC, z128, ones128, cnt_hbm, agg_hbm,
               acc_sh, idx_s, idx_d, rows0, rows1, sem0, sem1):
    c = lax.axis_index("c")
    s = lax.axis_index("s")
    r0 = s * RPTH
    pltpu.sync_copy(z128.at[pl.ds(r0, RPTH), :], acc_sh.at[pl.ds(r0, RPTH), :])
    pltpu.sync_copy(ones128, rows0)
    plsc.subcore_barrier()

    def outer_cnt(o, carry):
        pltpu.sync_copy(dstC.at[c, s, o], idx_d)
        cps = [pltpu.async_copy(rows0, acc_sh.at[idx_d.at[k]], sem0 if
                                k % 2 == 0 else sem1, add=True)
               for k in range(IC)]
        for cp in cps:
            cp.wait()
        return carry

    lax.fori_loop(0, OC, outer_cnt, 0)
    plsc.subcore_barrier()
    pltpu.sync_copy(acc_sh.at[pl.ds(r0, RPTH), :],
                    cnt_hbm.at[c, pl.ds(r0, RPTH), :])
    pltpu.sync_copy(z128.at[pl.ds(r0, RPTH), :], acc_sh.at[pl.ds(r0, RPTH), :])
    plsc.subcore_barrier()

    def outer_agg(o, carry):
        pltpu.sync_copy(srcT.at[s, o], idx_s)
        pltpu.sync_copy(dstC.at[c, s, o], idx_d)
        bufs = (rows0, rows1)
        sems = (sem0, sem1)
        g = pltpu.async_copy(x.at[idx_s.at[0]], bufs[0], sems[0])
        for k in range(IC):
            nxt = None
            if k + 1 < IC:
                nxt = pltpu.async_copy(x.at[idx_s.at[k + 1]],
                                       bufs[(k + 1) % 2], sems[(k + 1) % 2])
            g.wait()
            pltpu.sync_copy(bufs[k % 2], acc_sh.at[idx_d.at[k]], add=True)
            g = nxt
        return carry

    lax.fori_loop(0, OC, outer_agg, 0)
    plsc.subcore_barrier()
    pltpu.sync_copy(acc_sh.at[pl.ds(r0, RPTH), :],
                    agg_hbm.at[c, pl.ds(r0, RPTH), :])


@functools.partial(
    pl.kernel,
    out_type=jax.ShapeDtypeStruct((NC, NP, 128), _f32),
    mesh=plsc.VectorSubcoreMesh(core_axis_name="c", subcore_axis_name="s"),
    scratch_types=[
        pltpu.VMEM_SHARED((NP, 128), _f32),
        pltpu.VMEM((IC, B), jnp.int32),
        pltpu.VMEM((IC, B), jnp.int32),
        pltpu.VMEM((B, 128), _f32),
        pltpu.VMEM((B, 128), _f32),
        pltpu.SemaphoreType.DMA,
        pltpu.SemaphoreType.DMA,
    ],
)
def _sc_layer2(h1f, srcMc, dstT, z128, agg_hbm,
               agg_sh, idx_s, idx_d, rows0, rows1, sem0, sem1):
    c = lax.axis_index("c")
    s = lax.axis_index("s")
    r0 = s * RPT
    pltpu.sync_copy(z128.at[pl.ds(r0, RPT), :], agg_sh.at[pl.ds(r0, RPT), :])
    plsc.subcore_barrier()

    def outer(o, carry):
        pltpu.sync_copy(srcMc.at[c, s, o], idx_s)
        pltpu.sync_copy(dstT.at[s, o], idx_d)
        bufs = (rows0, rows1)
        sems = (sem0, sem1)
        g = pltpu.async_copy(h1f.at[idx_s.at[0]], bufs[0], sems[0])
        for k in range(IC):
            nxt = None
            if k + 1 < IC:
                nxt = pltpu.async_copy(h1f.at[idx_s.at[k + 1]],
                                       bufs[(k + 1) % 2], sems[(k + 1) % 2])
            g.wait()
            pltpu.sync_copy(bufs[k % 2], agg_sh.at[idx_d.at[k]], add=True)
            g = nxt
        return carry

    lax.fori_loop(0, OC, outer, 0)
    plsc.subcore_barrier()
    pltpu.sync_copy(agg_sh.at[pl.ds(r0, RPT), :],
                    agg_hbm.at[c, pl.ds(r0, RPT), :])


@functools.partial(
    pl.kernel,
    out_type=jax.ShapeDtypeStruct((E, 2 * H + DE), _f32),
    mesh=plsc.VectorSubcoreMesh(core_axis_name="c", subcore_axis_name="s"),
    scratch_types=[
        pltpu.VMEM((IC3, B3), jnp.int32),
        pltpu.VMEM((IC3, B3), jnp.int32),
        pltpu.VMEM((B3, H), _f32),
        pltpu.VMEM((B3, H), _f32),
        pltpu.VMEM((B3, H), _f32),
        pltpu.VMEM((B3, DE), _f32),
        pltpu.SemaphoreType.DMA,
        pltpu.SemaphoreType.DMA,
        pltpu.SemaphoreType.DMA,
        pltpu.SemaphoreType.DMA,
        pltpu.SemaphoreType.DMA,
        pltpu.SemaphoreType.DMA,
    ],
)
def _sc_out(g, srcW, dstW, ea, out_hbm,
            idx_s, idx_d, bs0, bs1, bd, be,
            ss0, ss1, sd, ts0, ts1, td):
    c = lax.axis_index("c")
    s = lax.axis_index("s")
    w = s * NC + c

    def outer(o, carry):
        pltpu.sync_copy(srcW.at[w, o], idx_s)
        pltpu.sync_copy(dstW.at[w, o], idx_d)
        bS = (bs0, bs1)
        sS = (ss0, ss1)
        tS = (ts0, ts1)
        st_s = [None, None]
        st_d = [None]
        gs = pltpu.async_copy(g.at[idx_s.at[0]], bS[0], ss0)
        for k in range(IC3):
            p = k % 2
            e0 = (w * OC * IC3 + o * IC3 + k) * B3
            if st_d[0] is not None:
                st_d[0].wait()
                st_d[0] = None
            gd = pltpu.async_copy(g.at[idx_d.at[k]], bd, sd)
            ns = None
            if k + 1 < IC3:
                q = (k + 1) % 2
                if st_s[q] is not None:
                    st_s[q].wait()
                    st_s[q] = None
                ns = pltpu.async_copy(g.at[idx_s.at[k + 1]], bS[q], sS[q])
            pltpu.sync_copy(ea.at[pl.ds(e0, B3), :], be)
            pltpu.sync_copy(be, out_hbm.at[pl.ds(e0, B3), pl.ds(2 * H, DE)])
            gs.wait()
            st_s[p] = pltpu.async_copy(
                bS[p], out_hbm.at[pl.ds(e0, B3), pl.ds(0, H)], tS[p])
            gd.wait()
            st_d[0] = pltpu.async_copy(
                bd, out_hbm.at[pl.ds(e0, B3), pl.ds(H, H)], td)
            gs = ns
        for par in (0, 1):
            if st_s[par] is not None:
                st_s[par].wait()
        if st_d[0] is not None:
            st_d[0].wait()
        return carry

    lax.fori_loop(0, OC, outer, 0)


def _agg_map(i):
    return (i // 5, i % 5, 0)


def _tc1_body(agg_ref, cnt_ref, x_ref, w1l_ref, b1_ref, w1r_ref, o_ref):
    inv = 1.0 / jnp.maximum(cnt_ref[0][:, 0:1], 1.0)
    h = (jnp.dot(agg_ref[0] * inv, w1l_ref[...], preferred_element_type=_f32)
         + jnp.dot(x_ref[...], w1r_ref[...], preferred_element_type=_f32)
         + b1_ref[...])
    h = jnp.maximum(h, 0.0)
    o_ref[0] = h[:, 0:128]
    o_ref[1] = h[:, 128:256]


def _tc1(agg1, cnt, x, w1l, b1, w1r):
    return pl.pallas_call(
        _tc1_body,
        grid=(N // BN,),
        in_specs=[
            pl.BlockSpec((1, BN, 128), _agg_map),
            pl.BlockSpec((1, BN, 128), _agg_map),
            pl.BlockSpec((BN, D), lambda i: (i, 0)),
            pl.BlockSpec((D, H), lambda i: (0, 0)),
            pl.BlockSpec((1, H), lambda i: (0, 0)),
            pl.BlockSpec((D, H), lambda i: (0, 0)),
        ],
        out_specs=pl.BlockSpec((NC, BN, 128), lambda i: (0, i, 0)),
        out_shape=jax.ShapeDtypeStruct((NC, N, 128), _f32),
    )(agg1, cnt, x, w1l, b1, w1r)


def _tc2_body(agg2_ref, cnt_ref, h1_ref, w2l_ref, b2_ref, w2r_ref, o_ref):
    inv = 1.0 / jnp.maximum(cnt_ref[0][:, 0:1], 1.0)
    t = (jnp.dot(agg2_ref[0] * inv, w2l_ref[0:128, :],
                 preferred_element_type=_f32)
         + jnp.dot(agg2_ref[1] * inv, w2l_ref[128:256, :],
                   preferred_element_type=_f32)
         + jnp.dot(h1_ref[0], w2r_ref[0:128, :], preferred_element_type=_f32)
         + jnp.dot(h1_ref[1], w2r_ref[128:256, :],
                   preferred_element_type=_f32)
         + b2_ref[...])
    o_ref[...] = 1.0 / (1.0 + jnp.exp(-t))


def _tc2(agg2, cnt, h1s, w2l, b2, w2r):
    return pl.pallas_call(
        _tc2_body,
        grid=(N // BN,),
        in_specs=[
            pl.BlockSpec((NC, BN, 128), lambda i: (0, i, 0)),
            pl.BlockSpec((1, BN, 128), _agg_map),
            pl.BlockSpec((NC, BN, 128), lambda i: (0, i, 0)),
            pl.BlockSpec((H, H), lambda i: (0, 0)),
            pl.BlockSpec((1, H), lambda i: (0, 0)),
            pl.BlockSpec((H, H), lambda i: (0, 0)),
        ],
        out_specs=pl.BlockSpec((BN, H), lambda i: (i, 0)),
        out_shape=jax.ShapeDtypeStruct((N, H), _f32),
    )(agg2, cnt, h1s, w2l, b2, w2r)


def _tc3_body(ea_ref, o_ref):
    o_ref[...] = 1.0 / (1.0 + jnp.exp(-ea_ref[...]))


def _tc3(edge_attr):
    return pl.pallas_call(
        _tc3_body,
        grid=(E // BE,),
        in_specs=[pl.BlockSpec((BE, DE), lambda i: (i, 0))],
        out_specs=pl.BlockSpec((BE, DE), lambda i: (i, 0)),
        out_shape=jax.ShapeDtypeStruct((E, DE), _f32),
    )(edge_attr)


def kernel(x, edge_index, edge_attr, W1l, b1, W1r, W2l, b2, W2r):
    src = edge_index[0]
    dst = edge_index[1]
    srcT = src.reshape(NS, OC, IC, B)
    srcMc = jnp.stack([srcT, srcT + N])
    dstT = dst.reshape(NS, OC, IC, B)
    dstC = jnp.stack([jnp.where(dst < NH, dst, JUNK),
                      jnp.where(dst >= NH, dst - NH, JUNK)])
    dstC = dstC.reshape(NC, NS, OC, IC, B)
    srcW = src.reshape(NC * NS, OC, IC3, B3)
    dstW = dst.reshape(NC * NS, OC, IC3, B3)
    z128 = jnp.zeros((NP, 128), _f32)
    ones128 = jnp.ones((B, 128), _f32)

    cnt, agg1 = _sc_layer1(x, srcT, dstC, z128, ones128)
    h1s = _tc1(agg1, cnt, x, W1l, b1.reshape(1, H), W1r)
    agg2 = _sc_layer2(h1s.reshape(2 * N, 128), srcMc, dstT, z128)
    g = _tc2(agg2, cnt, h1s, W2l, b2.reshape(1, H), W2r)
    ea = _tc3(edge_attr)
    out = _sc_out(g, srcW, dstW, ea)
    return out
